# baseline (speedup 1.0000x reference)
import functools

import jax
import jax.numpy as jnp
from jax import lax
from jax.experimental import pallas as pl
from jax.experimental.pallas import tpu as pltpu
from jax.experimental.pallas import tpu_sc as plsc

VOCAB = 1000000
D = 64
B = 4096
S = 200
TOTAL = B * S

NC, NS = 2, 16
NW = NC * NS
PER_W = TOTAL // NW
CHUNK = 128
NCHUNK = PER_W // CHUNK


def _gather_body(x_hbm, w_hbm, out_hbm, idx_v, rows_v, sem):
    wid = lax.axis_index("s") * NC + lax.axis_index("c")
    pltpu.sync_copy(x_hbm.at[wid], idx_v)
    base = wid * PER_W

    @pl.loop(0, NCHUNK)
    def _(g):
        pltpu.async_copy(w_hbm.at[idx_v.at[g]], rows_v, sem).wait()
        pltpu.sync_copy(rows_v, out_hbm.at[pl.ds(base + g * CHUNK, CHUNK)])


@jax.jit
def kernel(x, W):
    x_flat = x.reshape(NW, NCHUNK, CHUNK).astype(jnp.int32)
    mesh = plsc.VectorSubcoreMesh(
        core_axis_name="c", subcore_axis_name="s",
        num_cores=NC, num_subcores=NS)
    out = pl.kernel(
        _gather_body,
        out_type=jax.ShapeDtypeStruct((TOTAL, D), jnp.float32),
        mesh=mesh,
        compiler_params=pltpu.CompilerParams(use_tc_tiling_on_sc=False),
        scratch_types=[
            pltpu.VMEM((NCHUNK, CHUNK), jnp.int32),
            pltpu.VMEM((CHUNK, D), jnp.float32),
            pltpu.SemaphoreType.DMA,
        ],
    )(x_flat, W)
    return out.reshape(B, S, D)


# speedup vs baseline: 1.1191x; 1.1191x over previous
import functools

import jax
import jax.numpy as jnp
from jax import lax
from jax.experimental import pallas as pl
from jax.experimental.pallas import tpu as pltpu
from jax.experimental.pallas import tpu_sc as plsc

VOCAB = 1000000
D = 64
B = 4096
S = 200
TOTAL = B * S

NC, NS = 2, 16
NW = NC * NS
PER_W = TOTAL // NW
CHUNK = 128
NCHUNK = PER_W // CHUNK


NBUF = 4
NROUND = NCHUNK // NBUF


def _gather_body(x_hbm, w_hbm, out_hbm, idx_v, rows_v, gsems, osems):
    wid = lax.axis_index("s") * NC + lax.axis_index("c")
    pltpu.sync_copy(x_hbm.at[wid], idx_v)
    base = wid * PER_W

    def start_gather(g, b):
        pltpu.async_copy(w_hbm.at[idx_v.at[g]], rows_v.at[b], gsems.at[b])

    def wait_gather(g, b):
        pltpu.make_async_copy(w_hbm.at[idx_v.at[g]], rows_v.at[b],
                              gsems.at[b]).wait()

    def start_out(g, b):
        pltpu.async_copy(rows_v.at[b],
                         out_hbm.at[pl.ds(base + g * CHUNK, CHUNK)], osems.at[b])

    def wait_out(g, b):
        pltpu.make_async_copy(rows_v.at[b],
                              out_hbm.at[pl.ds(base + g * CHUNK, CHUNK)],
                              osems.at[b]).wait()

    for b in range(NBUF):
        start_gather(b, b)

    @pl.loop(0, NROUND - 1)
    def _(t):
        for b in range(NBUF):
            g = t * NBUF + b
            wait_gather(g, b)
            start_out(g, b)
            wait_out(g, b)
            start_gather(g + NBUF, b)

    for b in range(NBUF):
        g = (NROUND - 1) * NBUF + b
        wait_gather(g, b)
        start_out(g, b)
    for b in range(NBUF):
        g = (NROUND - 1) * NBUF + b
        wait_out(g, b)


@jax.jit
def kernel(x, W):
    x_flat = x.reshape(NW, NCHUNK, CHUNK).astype(jnp.int32)
    mesh = plsc.VectorSubcoreMesh(
        core_axis_name="c", subcore_axis_name="s",
        num_cores=NC, num_subcores=NS)
    out = pl.kernel(
        _gather_body,
        out_type=jax.ShapeDtypeStruct((TOTAL, D), jnp.float32),
        mesh=mesh,
        compiler_params=pltpu.CompilerParams(use_tc_tiling_on_sc=False),
        scratch_types=[
            pltpu.VMEM((NCHUNK, CHUNK), jnp.int32),
            pltpu.VMEM((NBUF, CHUNK, D), jnp.float32),
            pltpu.SemaphoreType.DMA((NBUF,)),
            pltpu.SemaphoreType.DMA((NBUF,)),
        ],
    )(x_flat, W)
    return out.reshape(B, S, D)
